# baseline (speedup 1.0000x reference)
import jax
import jax.numpy as jnp
from jax import lax
from jax.experimental import pallas as pl
from jax.experimental.pallas import tpu as pltpu

NBINS = 256
B, C, H, W, D = 4, 32, 64, 64, 64
ROWS = B * C
N = H * W * D
CENTER_H = (N // 2) // (W * D)
NORM = 65 * 65 * 65
LOG2E = 1.4426950408889634

NSTREAM = 4
RPB = 1
SHARE = ROWS // NSTREAM
STEPS = SHARE // RPB


def _entropy_body(*refs):
    in_refs = refs[:NSTREAM]
    out_ref, acc_ref, cen_ref = refs[NSTREAM], refs[NSTREAM + 1], refs[NSTREAM + 2]
    i = pl.program_id(0)

    for k, ref in enumerate(in_refs):
        for r in range(RPB):
            row = i * NSTREAM + k
            blk = ref[0, r]
            s = blk.sum(axis=0).sum(axis=0, keepdims=True)
            acc_ref[pl.ds(row, 1), :] = s
            cen_ref[pl.ds(row, 1), :] = blk[CENTER_H, 0:1, 0:1]

    @pl.when(i == STEPS - 1)
    def _():
        sums = acc_ref[...].sum(axis=1, keepdims=True)
        cen = cen_ref[...]
        nb = (sums - cen) * (1.0 / (N - 1))
        vals = jnp.concatenate([cen, nb], axis=0)
        idx = jnp.floor(vals * NBINS).astype(jnp.int32)
        valid = (vals >= 0.0) & (vals <= 1.0)
        idx = jnp.minimum(idx, NBINS - 1)
        bins = lax.broadcasted_iota(jnp.int32, (2 * ROWS, NBINS), 1)
        match = (idx == bins) & valid
        counts = jnp.sum(match.astype(jnp.float32), axis=0, keepdims=True)
        p = counts * (1.0 / NORM)
        e = -jnp.sum(p * (jnp.log(p + 1e-10) * LOG2E), axis=1, keepdims=True)
        out_ref[...] = e


def kernel(F):
    out = pl.pallas_call(
        _entropy_body,
        grid=(STEPS,),
        in_specs=[
            pl.BlockSpec(
                (1, RPB, H, W, D),
                (lambda i, _k=k: ((i * NSTREAM + _k) // C,
                                  (i * NSTREAM + _k) % C,
                                  0, 0, 0)),
            )
            for k in range(NSTREAM)
        ],
        out_specs=pl.BlockSpec((1, 1), lambda i: (0, 0)),
        out_shape=jax.ShapeDtypeStruct((1, 1), jnp.float32),
        scratch_shapes=[
            pltpu.VMEM((ROWS, D), jnp.float32),
            pltpu.VMEM((ROWS, 1), jnp.float32),
        ],
    )(*([F] * NSTREAM))
    return out.reshape(())
